# Initial kernel scaffold; baseline (speedup 1.0000x reference)
import jax
import jax.numpy as jnp
from jax.experimental import pallas as pl


def kernel(v, e, edge_index, v1_size, W1_1, b1_1, W1_2, b1_2, ln_w, ln_b):
    raise NotImplementedError("write your pallas kernel here")


# speedup vs baseline: 6.0041x; 6.0041x over previous
import functools

import jax
import jax.numpy as jnp
import numpy as np
from jax import lax
from jax.experimental import pallas as pl
from jax.experimental.pallas import tpu as pltpu
from jax.experimental.pallas import tpu_sc as plsc

N = 10000
E = 320000
H = 128
NF = 128
SHIFT = float(np.log(2.0))

NC = 2
NS = 16
NW = NC * NS
PER_W = E // NW
CHUNK = 80
NCH = PER_W // CHUNK
NPAD = 10240
ROWS_PER_TILE = NPAD // NS


def _sc_segment_sum(e, dst3, zeros):
    mesh = plsc.VectorSubcoreMesh(core_axis_name="c", subcore_axis_name="s")

    @functools.partial(
        pl.kernel,
        out_type=jax.ShapeDtypeStruct((NC, NPAD, H), jnp.float32),
        mesh=mesh,
        scratch_types=[
            pltpu.VMEM_SHARED((NPAD, H), jnp.float32),
            pltpu.VMEM((2, CHUNK, H), jnp.float32),
            pltpu.VMEM((NCH, CHUNK), jnp.int32),
            pltpu.SemaphoreType.DMA((2,)),
        ],
    )
    def seg(e_hbm, dst_hbm, zero_hbm, out_hbm, acc, rows, idx, sem):
        cid = lax.axis_index("c")
        sid = lax.axis_index("s")
        w = cid * NS + sid
        ebase = w * PER_W

        pltpu.sync_copy(zero_hbm, acc.at[pl.ds(sid * ROWS_PER_TILE, ROWS_PER_TILE)])
        pltpu.sync_copy(dst_hbm.at[w], idx)
        plsc.subcore_barrier()

        def start(j, slot):
            pltpu.async_copy(
                e_hbm.at[pl.ds(ebase + j * CHUNK, CHUNK)], rows.at[slot], sem.at[slot]
            )

        start(0, 0)

        def body(j, carry):
            slot = lax.rem(j, 2)

            @pl.when(j + 1 < NCH)
            def _():
                start(j + 1, 1 - slot)

            pltpu.make_async_copy(
                e_hbm.at[pl.ds(ebase + j * CHUNK, CHUNK)], rows.at[slot], sem.at[slot]
            ).wait()
            pltpu.sync_copy(rows.at[slot], acc.at[idx.at[j]], add=True)
            return carry

        lax.fori_loop(0, NCH, body, 0)
        plsc.subcore_barrier()

        r0 = sid * ROWS_PER_TILE
        pltpu.sync_copy(
            acc.at[pl.ds(r0, ROWS_PER_TILE)], out_hbm.at[cid, pl.ds(r0, ROWS_PER_TILE)]
        )

    return seg(e, dst3, zeros)


BLK = 1000


def _tc_body(p_ref, v_ref, w1_ref, b1_ref, w2_ref, b2_ref, lnw_ref, lnb_ref, out_ref):
    x = p_ref[0] + p_ref[1]
    h = jnp.dot(x, w1_ref[...], preferred_element_type=jnp.float32,
                precision=lax.Precision.HIGHEST) + b1_ref[...]
    s = jnp.maximum(h, 0.0) + jnp.log1p(jnp.exp(-jnp.abs(h))) - SHIFT
    y = jnp.dot(s, w2_ref[...], preferred_element_type=jnp.float32,
                precision=lax.Precision.HIGHEST) + b2_ref[...]
    mu = jnp.mean(y, axis=-1, keepdims=True)
    yc = y - mu
    var = jnp.mean(yc * yc, axis=-1, keepdims=True)
    out_ref[...] = v_ref[...] + yc * lax.rsqrt(var + 1e-5) * lnw_ref[...] + lnb_ref[...]


def _tc_mlp(partials, v, W1, b1, W2, b2, lnw, lnb):
    return pl.pallas_call(
        _tc_body,
        grid=(N // BLK,),
        in_specs=[
            pl.BlockSpec((NC, BLK, H), lambda i: (0, i, 0)),
            pl.BlockSpec((BLK, H), lambda i: (i, 0)),
            pl.BlockSpec((H, H), lambda i: (0, 0)),
            pl.BlockSpec((1, H), lambda i: (0, 0)),
            pl.BlockSpec((H, H), lambda i: (0, 0)),
            pl.BlockSpec((1, H), lambda i: (0, 0)),
            pl.BlockSpec((1, H), lambda i: (0, 0)),
            pl.BlockSpec((1, H), lambda i: (0, 0)),
        ],
        out_specs=pl.BlockSpec((BLK, H), lambda i: (i, 0)),
        out_shape=jax.ShapeDtypeStruct((N, H), jnp.float32),
    )(partials, v, W1, b1, W2, b2, lnw, lnb)


def kernel(v, e, edge_index, v1_size, W1_1, b1_1, W1_2, b1_2, ln_w, ln_b):
    del v1_size
    dst3 = edge_index[1].reshape(NW, NCH, CHUNK)
    zeros = jnp.zeros((ROWS_PER_TILE, H), jnp.float32)
    partials = _sc_segment_sum(e, dst3, zeros)
    return _tc_mlp(
        partials, v, W1_1, b1_1.reshape(1, H), W1_2, b1_2.reshape(1, H),
        ln_w.reshape(1, H), ln_b.reshape(1, H),
    )
